# baseline (speedup 1.0000x reference)
import functools

import jax
import jax.numpy as jnp
import numpy as np
from jax import lax
from jax.experimental import pallas as pl
from jax.experimental.pallas import tpu as pltpu
from jax.experimental.pallas import tpu_sc as plsc

C = 384
H = W = 384
HW = H * W
NM = 1024
NNM = NM * 150
K = NM + NNM
NC = 2
NS = 16
NW = NC * NS
PB = 4864
K_PAD = NW * PB
CH = 64
NCH = PB // CH


def _tr_body(a_ref, b_ref, a0, a1, b0, b1):
    zeros = jnp.zeros((128, 128), jnp.float32)
    for img_ref, outs in ((a_ref, (a0, a1)), (b_ref, (b0, b1))):
        for h in range(8):
            blk = img_ref[0, :, h, :]
            p01 = pltpu.pack_elementwise(
                [blk[0:128], blk[128:256]], packed_dtype=jnp.bfloat16
            )
            p2 = pltpu.pack_elementwise(
                [blk[256:384], zeros], packed_dtype=jnp.bfloat16
            )
            sl = slice(h * 128, (h + 1) * 128)
            outs[0][sl, :] = lax.bitcast_convert_type(p01, jnp.float32).T
            outs[1][sl, :] = lax.bitcast_convert_type(p2, jnp.float32).T


_transpose = pl.pallas_call(
    _tr_body,
    grid=((H // 8) * (W // 128),),
    in_specs=[
        pl.BlockSpec((1, C, 8, 128), lambda i: (0, 0, i // 3, i % 3)),
        pl.BlockSpec((1, C, 8, 128), lambda i: (0, 0, i // 3, i % 3)),
    ],
    out_specs=[pl.BlockSpec((1024, 128), lambda i: (i, 0))] * 4,
    out_shape=[jax.ShapeDtypeStruct((HW, 128), jnp.float32)] * 4,
)

_mesh = plsc.VectorSubcoreMesh(
    core_axis_name="c", subcore_axis_name="s", num_cores=NC, num_subcores=NS
)


@functools.partial(
    pl.kernel,
    out_type=jax.ShapeDtypeStruct((NW, 2, 16), jnp.float32),
    mesh=_mesh,
    scratch_types=[
        pltpu.VMEM((PB,), jnp.int32),
        pltpu.VMEM((PB,), jnp.int32),
    ]
    + [pltpu.VMEM((2, CH, 128), jnp.float32)] * 4
    + [
        pltpu.VMEM((2, 16), jnp.float32),
        pltpu.SemaphoreType.DMA,
        pltpu.SemaphoreType.DMA,
    ],
    compiler_params=pltpu.CompilerParams(
        use_tc_tiling_on_sc=False, needs_layout_passes=False
    ),
)
def _sc_dist(a0_h, a1_h, b0_h, b1_h, ia_hbm, ib_hbm, out_hbm,
             ia_v, ib_v, a0_v, a1_v, b0_v, b1_v, acc_v, sem0, sem1):
    wid = lax.axis_index("s") * NC + lax.axis_index("c")
    base = wid * PB
    pltpu.sync_copy(ia_hbm.at[pl.ds(base, PB)], ia_v)
    pltpu.sync_copy(ib_hbm.at[pl.ds(base, PB)], ib_v)
    zero = jnp.zeros((16,), jnp.float32)

    a_tabs = (a0_h, a1_h)
    b_tabs = (b0_h, b1_h)
    a_bufs = (a0_v, a1_v)
    b_bufs = (b0_v, b1_v)
    sems = (sem0, sem1)

    def fire(ch, par):
        c0 = ch * CH
        ias = ia_v.at[pl.ds(c0, CH)]
        ibs = ib_v.at[pl.ds(c0, CH)]
        for tab, buf in zip(a_tabs, a_bufs):
            pltpu.async_copy(tab.at[ias], buf.at[par], sems[par])
        for tab, buf in zip(b_tabs, b_bufs):
            pltpu.async_copy(tab.at[ibs], buf.at[par], sems[par])

    def drain(par):
        for tab, buf in zip(a_tabs + b_tabs, a_bufs + b_bufs):
            pltpu.make_async_copy(
                tab.at[pl.ds(0, CH)], buf.at[par], sems[par]
            ).wait()

    def compute(ch, par, carry):
        gbase = base + ch * CH

        def pair_body(r, c2):
            m, n = c2
            accs = [zero, zero, zero, zero]
            cnt = 0
            for abuf, bbuf in zip(a_bufs, b_bufs):
                for j in range(8):
                    va = abuf[par, r, pl.ds(16 * j, 16)]
                    vb = bbuf[par, r, pl.ds(16 * j, 16)]
                    d = plsc.bitcast(va, jnp.bfloat16) - plsc.bitcast(
                        vb, jnp.bfloat16
                    )
                    dl, dh = plsc.unpack(d, format=plsc.PackFormat.INTERLEAVED)
                    accs[cnt % 2] = accs[cnt % 2] + dl * dl
                    accs[2 + cnt % 2] = accs[2 + cnt % 2] + dh * dh
                    cnt += 1
            acc = (accs[0] + accs[1]) + (accs[2] + accs[3])
            s = plsc.cumsum(acc)
            gv = jnp.full((16,), gbase + r, jnp.int32)
            is_m = gv < NM
            ok = gv < K
            m = m + jnp.where(is_m, s, 0.0)
            n = n + jnp.where(
                jnp.logical_and(ok, jnp.logical_not(is_m)),
                jnp.maximum(0.5 - s, 0.0),
                0.0,
            )
            return m, n

        return lax.fori_loop(0, CH, pair_body, carry, unroll=4)

    fire(0, 0)

    def body(i, carry):
        ch0 = 2 * i
        fire(ch0 + 1, 1)
        drain(0)
        carry = compute(ch0, 0, carry)

        @pl.when(i < NCH // 2 - 1)
        def _():
            fire(ch0 + 2, 0)

        drain(1)
        carry = compute(ch0 + 1, 1, carry)
        return carry

    m_acc, n_acc = lax.fori_loop(0, NCH // 2, body, (zero, zero))
    acc_v[0] = m_acc
    acc_v[1] = n_acc
    pltpu.sync_copy(acc_v, out_hbm.at[wid])


def kernel(image_a_pred, image_b_pred, matches_a, matches_b,
           non_matches_a, non_matches_b):
    a0, a1, b0, b1 = _transpose(image_a_pred, image_b_pred)

    def pix(rc):
        r = rc[:, 0].astype(jnp.int32)
        c = rc[:, 1].astype(jnp.int32)
        return (r >> 3) * 3072 + ((c >> 7) << 10) + ((r & 7) << 7) + (c & 127)

    pad = jnp.zeros((K_PAD - K,), jnp.int32)
    ia = jnp.concatenate([pix(matches_a), pix(non_matches_a), pad])
    ib = jnp.concatenate([pix(matches_b), pix(non_matches_b), pad])

    out = _sc_dist(a0, a1, b0, b1, ia, ib)
    match_loss = jnp.sum(out[:, 0, 15]) / NM
    non_match_loss = jnp.sum(out[:, 1, 15]) / NNM
    loss = match_loss + non_match_loss
    return (loss, match_loss, non_match_loss)


# speedup vs baseline: 1.2528x; 1.2528x over previous
import functools

import jax
import jax.numpy as jnp
import numpy as np
from jax import lax
from jax.experimental import pallas as pl
from jax.experimental.pallas import tpu as pltpu
from jax.experimental.pallas import tpu_sc as plsc

C = 384
H = W = 384
HW = H * W
NM = 1024
NNM = NM * 150
K = NM + NNM
NC = 2
NS = 16
NW = NC * NS
PB = 4864
K_PAD = NW * PB
CH = 64
NCH = PB // CH


def _tr_body(a_ref, b_ref, a0, a1, b0, b1):
    zeros = jnp.zeros((128, 128), jnp.float32)
    for img_ref, outs in ((a_ref, (a0, a1)), (b_ref, (b0, b1))):
        for h in range(8):
            blk = img_ref[0, :, h, :]
            p01 = pltpu.pack_elementwise(
                [blk[0:128], blk[128:256]], packed_dtype=jnp.bfloat16
            )
            p2 = pltpu.pack_elementwise(
                [blk[256:384], zeros], packed_dtype=jnp.bfloat16
            )
            sl = slice(h * 128, (h + 1) * 128)
            outs[0][sl, :] = lax.bitcast_convert_type(p01, jnp.float32).T
            outs[1][sl, :] = lax.bitcast_convert_type(p2, jnp.float32).T


_transpose = pl.pallas_call(
    _tr_body,
    grid=((H // 8) * (W // 128),),
    in_specs=[
        pl.BlockSpec((1, C, 8, 128), lambda i: (0, 0, i // 3, i % 3)),
        pl.BlockSpec((1, C, 8, 128), lambda i: (0, 0, i // 3, i % 3)),
    ],
    out_specs=[pl.BlockSpec((1024, 128), lambda i: (i, 0))] * 4,
    out_shape=[jax.ShapeDtypeStruct((HW, 128), jnp.float32)] * 4,
)

_mesh = plsc.VectorSubcoreMesh(
    core_axis_name="c", subcore_axis_name="s", num_cores=NC, num_subcores=NS
)


@functools.partial(
    pl.kernel,
    out_type=jax.ShapeDtypeStruct((NW, 2, 16), jnp.float32),
    mesh=_mesh,
    scratch_types=[
        pltpu.VMEM((PB,), jnp.int32),
        pltpu.VMEM((PB,), jnp.int32),
    ]
    + [pltpu.VMEM((2, CH, 128), jnp.float32)] * 4
    + [
        pltpu.VMEM((2, 16), jnp.float32),
        pltpu.SemaphoreType.DMA,
        pltpu.SemaphoreType.DMA,
    ],
    compiler_params=pltpu.CompilerParams(
        use_tc_tiling_on_sc=False, needs_layout_passes=False
    ),
)
def _sc_dist(a0_h, a1_h, b0_h, b1_h, ia_hbm, ib_hbm, out_hbm,
             ia_v, ib_v, a0_v, a1_v, b0_v, b1_v, acc_v, sem0, sem1):
    wid = lax.axis_index("s") * NC + lax.axis_index("c")
    base = wid * PB
    pltpu.sync_copy(ia_hbm.at[pl.ds(base, PB)], ia_v)
    pltpu.sync_copy(ib_hbm.at[pl.ds(base, PB)], ib_v)
    zero = jnp.zeros((16,), jnp.float32)

    a_tabs = (a0_h, a1_h)
    b_tabs = (b0_h, b1_h)
    a_bufs = (a0_v, a1_v)
    b_bufs = (b0_v, b1_v)
    sems = (sem0, sem1)

    def fire(ch, par):
        c0 = ch * CH
        ias = ia_v.at[pl.ds(c0, CH)]
        ibs = ib_v.at[pl.ds(c0, CH)]
        for tab, buf in zip(a_tabs, a_bufs):
            pltpu.async_copy(tab.at[ias], buf.at[par], sems[par])
        for tab, buf in zip(b_tabs, b_bufs):
            pltpu.async_copy(tab.at[ibs], buf.at[par], sems[par])

    def drain(par):
        for tab, buf in zip(a_tabs + b_tabs, a_bufs + b_bufs):
            pltpu.make_async_copy(
                tab.at[pl.ds(0, CH)], buf.at[par], sems[par]
            ).wait()

    def compute(ch, par, carry):
        gbase = base + ch * CH

        def pair_body(r, c2):
            m, n = c2
            accs = [zero, zero, zero, zero]
            cnt = 0
            for abuf, bbuf in zip(a_bufs, b_bufs):
                for j in range(8):
                    va = abuf[par, r, pl.ds(16 * j, 16)]
                    vb = bbuf[par, r, pl.ds(16 * j, 16)]
                    d = plsc.bitcast(va, jnp.bfloat16) - plsc.bitcast(
                        vb, jnp.bfloat16
                    )
                    dl, dh = plsc.unpack(d, format=plsc.PackFormat.INTERLEAVED)
                    accs[cnt % 2] = accs[cnt % 2] + dl * dl
                    accs[2 + cnt % 2] = accs[2 + cnt % 2] + dh * dh
                    cnt += 1
            acc = (accs[0] + accs[1]) + (accs[2] + accs[3])
            s = plsc.cumsum(acc)
            gv = jnp.full((16,), gbase + r, jnp.int32)
            is_m = gv < NM
            ok = gv < K
            m = m + jnp.where(is_m, s, 0.0)
            n = n + jnp.where(
                jnp.logical_and(ok, jnp.logical_not(is_m)),
                jnp.maximum(0.5 - s, 0.0),
                0.0,
            )
            return m, n

        return lax.fori_loop(0, CH, pair_body, carry, unroll=2)

    fire(0, 0)

    def body(i, carry):
        ch0 = 2 * i
        fire(ch0 + 1, 1)
        drain(0)
        carry = compute(ch0, 0, carry)

        @pl.when(i < NCH // 2 - 1)
        def _():
            fire(ch0 + 2, 0)

        drain(1)
        carry = compute(ch0 + 1, 1, carry)
        return carry

    m_acc, n_acc = lax.fori_loop(0, NCH // 2, body, (zero, zero))
    acc_v[0] = m_acc
    acc_v[1] = n_acc
    pltpu.sync_copy(acc_v, out_hbm.at[wid])


def kernel(image_a_pred, image_b_pred, matches_a, matches_b,
           non_matches_a, non_matches_b):
    a0, a1, b0, b1 = _transpose(image_a_pred, image_b_pred)

    def pix(rc):
        r = rc[:, 0].astype(jnp.int32)
        c = rc[:, 1].astype(jnp.int32)
        return (r >> 3) * 3072 + ((c >> 7) << 10) + ((r & 7) << 7) + (c & 127)

    pad = jnp.zeros((K_PAD - K,), jnp.int32)
    ia = jnp.concatenate([pix(matches_a), pix(non_matches_a), pad])
    ib = jnp.concatenate([pix(matches_b), pix(non_matches_b), pad])

    out = _sc_dist(a0, a1, b0, b1, ia, ib)
    match_loss = jnp.sum(out[:, 0, 15]) / NM
    non_match_loss = jnp.sum(out[:, 1, 15]) / NNM
    loss = match_loss + non_match_loss
    return (loss, match_loss, non_match_loss)


# speedup vs baseline: 1.2846x; 1.0254x over previous
import functools

import jax
import jax.numpy as jnp
from jax import lax
from jax.experimental import pallas as pl
from jax.experimental.pallas import tpu as pltpu
from jax.experimental.pallas import tpu_sc as plsc

C = 384
H = W = 384
HW = H * W
NM = 1024
NNM = NM * 150
K = NM + NNM
NC = 2
NS = 16
NW = NC * NS
PB = 4864
K_PAD = NW * PB
CH = 64
NCH = PB // CH


def _tr_body(a_ref, b_ref, a0, a1, b0, b1):
    zeros = jnp.zeros((128, 128), jnp.float32)
    for img_ref, outs in ((a_ref, (a0, a1)), (b_ref, (b0, b1))):
        for h in range(8):
            blk = img_ref[0, :, h, :]
            p01 = pltpu.pack_elementwise(
                [blk[0:128], blk[128:256]], packed_dtype=jnp.bfloat16
            )
            p2 = pltpu.pack_elementwise(
                [blk[256:384], zeros], packed_dtype=jnp.bfloat16
            )
            sl = slice(h * 128, (h + 1) * 128)
            outs[0][sl, :] = lax.bitcast_convert_type(p01, jnp.float32).T
            outs[1][sl, :] = lax.bitcast_convert_type(p2, jnp.float32).T


_transpose = pl.pallas_call(
    _tr_body,
    grid=((H // 8) * (W // 128),),
    in_specs=[
        pl.BlockSpec((1, C, 8, 128), lambda i: (0, 0, i // 3, i % 3)),
        pl.BlockSpec((1, C, 8, 128), lambda i: (0, 0, i // 3, i % 3)),
    ],
    out_specs=[pl.BlockSpec((1024, 128), lambda i: (i, 0))] * 4,
    out_shape=[jax.ShapeDtypeStruct((HW, 128), jnp.float32)] * 4,
)

_mesh = plsc.VectorSubcoreMesh(
    core_axis_name="c", subcore_axis_name="s", num_cores=NC, num_subcores=NS
)


@functools.partial(
    pl.kernel,
    out_type=jax.ShapeDtypeStruct((NW, 2, 16), jnp.float32),
    mesh=_mesh,
    scratch_types=[
        pltpu.VMEM((PB,), jnp.int32),
        pltpu.VMEM((PB,), jnp.int32),
    ]
    + [pltpu.VMEM((3, CH, 128), jnp.float32)] * 4
    + [
        pltpu.VMEM((2, 16), jnp.float32),
        pltpu.SemaphoreType.DMA,
        pltpu.SemaphoreType.DMA,
        pltpu.SemaphoreType.DMA,
    ],
    compiler_params=pltpu.CompilerParams(
        use_tc_tiling_on_sc=False, needs_layout_passes=False
    ),
)
def _sc_dist(a0_h, a1_h, b0_h, b1_h, ia_hbm, ib_hbm, out_hbm,
             ia_v, ib_v, a0_v, a1_v, b0_v, b1_v, acc_v, sem0, sem1, sem2):
    wid = lax.axis_index("s") * NC + lax.axis_index("c")
    base = wid * PB
    pltpu.sync_copy(ia_hbm.at[pl.ds(base, PB)], ia_v)
    pltpu.sync_copy(ib_hbm.at[pl.ds(base, PB)], ib_v)
    zero = jnp.zeros((16,), jnp.float32)

    a_tabs = (a0_h, a1_h)
    b_tabs = (b0_h, b1_h)
    a_bufs = (a0_v, a1_v)
    b_bufs = (b0_v, b1_v)
    sems = (sem0, sem1, sem2)

    def fire(ch, par):
        c0 = ch * CH
        ias = ia_v.at[pl.ds(c0, CH)]
        ibs = ib_v.at[pl.ds(c0, CH)]
        for tab, buf in zip(a_tabs, a_bufs):
            pltpu.async_copy(tab.at[ias], buf.at[par], sems[par])
        for tab, buf in zip(b_tabs, b_bufs):
            pltpu.async_copy(tab.at[ibs], buf.at[par], sems[par])

    def drain(par):
        for tab, buf in zip(a_tabs + b_tabs, a_bufs + b_bufs):
            pltpu.make_async_copy(
                tab.at[pl.ds(0, CH)], buf.at[par], sems[par]
            ).wait()

    def compute(ch, par, carry):
        gbase = base + ch * CH
        wm = jnp.full((16,), jnp.where(gbase < NM, 1.0, 0.0), jnp.float32)
        wn = jnp.full(
            (16,),
            jnp.where(
                jnp.logical_and(gbase >= NM, gbase < K), 1.0, 0.0
            ),
            jnp.float32,
        )

        def pair_body(r, c2):
            m, n = c2
            accs = [zero, zero, zero, zero]
            cnt = 0
            for abuf, bbuf in zip(a_bufs, b_bufs):
                for j in range(8):
                    va = abuf[par, r, pl.ds(16 * j, 16)]
                    vb = bbuf[par, r, pl.ds(16 * j, 16)]
                    d = plsc.bitcast(va, jnp.bfloat16) - plsc.bitcast(
                        vb, jnp.bfloat16
                    )
                    dl, dh = plsc.unpack(d, format=plsc.PackFormat.INTERLEAVED)
                    accs[cnt % 2] = accs[cnt % 2] + dl * dl
                    accs[2 + cnt % 2] = accs[2 + cnt % 2] + dh * dh
                    cnt += 1
            acc = (accs[0] + accs[1]) + (accs[2] + accs[3])
            s = plsc.cumsum(acc)
            m = m + s * wm
            n = n + jnp.maximum(0.5 - s, 0.0) * wn
            return m, n

        return lax.fori_loop(0, CH, pair_body, carry, unroll=1)

    fire(0, 0)
    fire(1, 1)

    def body(i, carry):
        ch0 = 3 * i
        fire(ch0 + 2, 2)
        drain(0)
        carry = compute(ch0, 0, carry)
        fire(ch0 + 3, 0)
        drain(1)
        carry = compute(ch0 + 1, 1, carry)

        @pl.when(i < NCH // 3 - 1)
        def _():
            fire(ch0 + 4, 1)

        drain(2)
        carry = compute(ch0 + 2, 2, carry)
        return carry

    m_acc, n_acc = lax.fori_loop(0, NCH // 3, body, (zero, zero))
    drain(0)
    m_acc, n_acc = compute(NCH - 1, 0, (m_acc, n_acc))
    acc_v[0] = m_acc
    acc_v[1] = n_acc
    pltpu.sync_copy(acc_v, out_hbm.at[wid])


def kernel(image_a_pred, image_b_pred, matches_a, matches_b,
           non_matches_a, non_matches_b):
    a0, a1, b0, b1 = _transpose(image_a_pred, image_b_pred)

    def pix(rc):
        r = rc[:, 0].astype(jnp.int32)
        c = rc[:, 1].astype(jnp.int32)
        return (r >> 3) * 3072 + ((c >> 7) << 10) + ((r & 7) << 7) + (c & 127)

    pad = jnp.zeros((K_PAD - K,), jnp.int32)
    ia = jnp.concatenate([pix(matches_a), pix(non_matches_a), pad])
    ib = jnp.concatenate([pix(matches_b), pix(non_matches_b), pad])

    out = _sc_dist(a0, a1, b0, b1, ia, ib)
    match_loss = jnp.sum(out[:, 0, 15]) / NM
    non_match_loss = jnp.sum(out[:, 1, 15]) / NNM
    loss = match_loss + non_match_loss
    return (loss, match_loss, non_match_loss)
